# baseline (speedup 1.0000x reference)
import functools

import jax
import jax.numpy as jnp
from jax.experimental import pallas as pl
from jax.experimental.pallas import tpu as pltpu


def _dot(a, b):
    return jax.lax.dot_general(
        a, b, (((1,), (0,)), ((), ())),
        preferred_element_type=jnp.float32,
    )


def _att_pool_kernel(feat_ref, valid_ref,
                     fWi, fbi, fWb, fbb, fWs, fbs, fWa, fba,
                     mWi, mbi, mWb, mbb, mWs, mbs, mWa, mba,
                     out_ref, *, nb, kk, cc):
    x3 = feat_ref[...]
    x = x3.reshape(nb * kk, cc)

    h = jax.nn.relu(_dot(x, fWi[...]) + fbi[...])
    h = jax.nn.relu(_dot(h, fWb[...]) + fbb[...])
    h = h + jax.nn.relu(_dot(x, fWs[...]) + fbs[...])
    a = jax.nn.relu(_dot(h, fWa[...]) + fba[...])

    a3 = a.reshape(nb, kk, cc) * valid_ref[0][:, :, None]

    e = jnp.exp(a3)
    s = jnp.sum(e, axis=1)
    w = jnp.sum(x3 * e, axis=1)
    f_agg = w / (s * (1.0 + 1e-5))

    g = jax.nn.relu(_dot(f_agg, mWi[...]) + mbi[...])
    g = jax.nn.relu(_dot(g, mWb[...]) + mbb[...])
    g = g + jax.nn.relu(_dot(f_agg, mWs[...]) + mbs[...])
    out_ref[...] = jax.nn.relu(_dot(g, mWa[...]) + mba[...])


def kernel(feature_set, mask, fc_Wi, fc_bi, fc_Wb, fc_bb, fc_Ws, fc_bs, fc_Wa, fc_ba,
           mlp_Wi, mlp_bi, mlp_Wb, mlp_bb, mlp_Ws, mlp_bs, mlp_Wa, mlp_ba):
    N, K, C = feature_set.shape
    D_OUT = mlp_Wa.shape[1]
    NB = 400
    assert N % NB == 0

    valid = jnp.logical_not(mask).astype(jnp.float32).reshape(N // NB, NB, K)
    b2 = lambda b: b.reshape(1, -1)

    wspec = pl.BlockSpec((C, C), lambda i: (0, 0))
    waspec = pl.BlockSpec((C, D_OUT), lambda i: (0, 0))
    bspec = pl.BlockSpec((1, C), lambda i: (0, 0))
    baspec = pl.BlockSpec((1, D_OUT), lambda i: (0, 0))

    return pl.pallas_call(
        functools.partial(_att_pool_kernel, nb=NB, kk=K, cc=C),
        grid=(N // NB,),
        in_specs=[
            pl.BlockSpec((NB, K, C), lambda i: (i, 0, 0)),
            pl.BlockSpec((1, NB, K), lambda i: (i, 0, 0)),
            wspec, bspec, wspec, bspec, wspec, bspec, wspec, bspec,
            wspec, bspec, wspec, bspec, wspec, bspec, waspec, baspec,
        ],
        out_specs=pl.BlockSpec((NB, D_OUT), lambda i: (i, 0)),
        out_shape=jax.ShapeDtypeStruct((N, D_OUT), jnp.float32),
        compiler_params=pltpu.CompilerParams(
            dimension_semantics=("parallel",),
            vmem_limit_bytes=128 * 1024 * 1024),
    )(feature_set, valid,
      fc_Wi, b2(fc_bi), fc_Wb, b2(fc_bb), fc_Ws, b2(fc_bs), fc_Wa, b2(fc_ba),
      mlp_Wi, b2(mlp_bi), mlp_Wb, b2(mlp_bb), mlp_Ws, b2(mlp_bs), mlp_Wa, b2(mlp_ba))


# speedup vs baseline: 1.0711x; 1.0711x over previous
import functools

import jax
import jax.numpy as jnp
from jax.experimental import pallas as pl
from jax.experimental.pallas import tpu as pltpu


def _dot(a, b):
    return jax.lax.dot_general(
        a, b, (((1,), (0,)), ((), ())),
        preferred_element_type=jnp.float32,
    )


def _att_pool_kernel(feat_ref, valid_ref,
                     fWi, fbi, fWb, fbb, fWs, fbs, fWa, fba,
                     mWi, mbi, mWb, mbb, mWs, mbs, mWa, mba,
                     out_ref, *, nb, kk, cc):
    x3 = feat_ref[...]
    x = x3.reshape(nb * kk, cc)

    h = jax.nn.relu(_dot(x, fWi[...]) + fbi[...])
    h = jax.nn.relu(_dot(h, fWb[...]) + fbb[...])
    h = h + jax.nn.relu(_dot(x, fWs[...]) + fbs[...])
    a = jax.nn.relu(_dot(h, fWa[...]) + fba[...])

    a3 = a.reshape(nb, kk, cc) * valid_ref[0][:, :, None]

    e = jnp.exp2(a3)
    s = jnp.sum(e, axis=1)
    w = jnp.sum(x3 * e, axis=1)
    f_agg = w / (s * (1.0 + 1e-5))

    g = jax.nn.relu(_dot(f_agg, mWi[...]) + mbi[...])
    g = jax.nn.relu(_dot(g, mWb[...]) + mbb[...])
    g = g + jax.nn.relu(_dot(f_agg, mWs[...]) + mbs[...])
    out_ref[...] = jax.nn.relu(_dot(g, mWa[...]) + mba[...])


def kernel(feature_set, mask, fc_Wi, fc_bi, fc_Wb, fc_bb, fc_Ws, fc_bs, fc_Wa, fc_ba,
           mlp_Wi, mlp_bi, mlp_Wb, mlp_bb, mlp_Ws, mlp_bs, mlp_Wa, mlp_ba):
    N, K, C = feature_set.shape
    D_OUT = mlp_Wa.shape[1]
    NB = 1000
    assert N % NB == 0

    valid = jnp.where(mask, 0.0, 1.4426950408889634).reshape(N // NB, NB, K)
    b2 = lambda b: b.reshape(1, -1)

    wspec = pl.BlockSpec((C, C), lambda i: (0, 0))
    waspec = pl.BlockSpec((C, D_OUT), lambda i: (0, 0))
    bspec = pl.BlockSpec((1, C), lambda i: (0, 0))
    baspec = pl.BlockSpec((1, D_OUT), lambda i: (0, 0))

    return pl.pallas_call(
        functools.partial(_att_pool_kernel, nb=NB, kk=K, cc=C),
        grid=(N // NB,),
        in_specs=[
            pl.BlockSpec((NB, K, C), lambda i: (i, 0, 0)),
            pl.BlockSpec((1, NB, K), lambda i: (i, 0, 0)),
            wspec, bspec, wspec, bspec, wspec, bspec, wspec, bspec,
            wspec, bspec, wspec, bspec, wspec, bspec, waspec, baspec,
        ],
        out_specs=pl.BlockSpec((NB, D_OUT), lambda i: (i, 0)),
        out_shape=jax.ShapeDtypeStruct((N, D_OUT), jnp.float32),
        compiler_params=pltpu.CompilerParams(
            dimension_semantics=("parallel",),
            vmem_limit_bytes=128 * 1024 * 1024),
    )(feature_set, valid,
      fc_Wi, b2(fc_bi), fc_Wb, b2(fc_bb), fc_Ws, b2(fc_bs), fc_Wa, b2(fc_ba),
      mlp_Wi, b2(mlp_bi), mlp_Wb, b2(mlp_bb), mlp_Ws, b2(mlp_bs), mlp_Wa, b2(mlp_ba))
